# baseline (speedup 1.0000x reference)
import jax
import jax.numpy as jnp
from jax.experimental import pallas as pl
from jax.experimental.pallas import tpu as pltpu

_TM = 512


def _stage1_body(g_ref, x_ref, w1_ref, b1_ref, w2_ref, b2_ref,
                 p2_ref, g16_ref):
    g = g_ref[...]
    g16_ref[...] = g.astype(jnp.bfloat16)
    m1 = jnp.dot(g, x_ref[...], preferred_element_type=jnp.float32)
    rs = jnp.sum(g, axis=1, keepdims=True)
    h = jnp.dot(m1, w1_ref[...], preferred_element_type=jnp.float32)
    h = jnp.maximum(h + rs * b1_ref[...], 0.0)
    p2 = jnp.dot(h, w2_ref[...], preferred_element_type=jnp.float32)
    p2_ref[...] = (p2 + b2_ref[...]).astype(p2_ref.dtype)


def _stage2_body(g16_ref, p2_ref, o_ref):
    o_ref[...] = jnp.dot(g16_ref[...], p2_ref[...],
                         preferred_element_type=jnp.float32)


def kernel(x, G, w1, b1, w2, b2):
    N, C = x.shape
    H = w1.shape[1]
    K = w2.shape[1]
    tm = _TM
    b1r = b1.reshape(1, H)
    b2r = b2.reshape(1, K)

    p2, g16 = pl.pallas_call(
        _stage1_body,
        out_shape=(
            jax.ShapeDtypeStruct((N, K), jnp.bfloat16),
            jax.ShapeDtypeStruct((N, N), jnp.bfloat16),
        ),
        grid=(N // tm,),
        in_specs=[
            pl.BlockSpec((tm, N), lambda i: (i, 0)),
            pl.BlockSpec((N, C), lambda i: (0, 0)),
            pl.BlockSpec((C, H), lambda i: (0, 0)),
            pl.BlockSpec((1, H), lambda i: (0, 0)),
            pl.BlockSpec((H, K), lambda i: (0, 0)),
            pl.BlockSpec((1, K), lambda i: (0, 0)),
        ],
        out_specs=(
            pl.BlockSpec((tm, K), lambda i: (i, 0)),
            pl.BlockSpec((tm, N), lambda i: (i, 0)),
        ),
        compiler_params=pltpu.CompilerParams(
            dimension_semantics=("parallel",),
            vmem_limit_bytes=52 * 1024 * 1024,
        ),
    )(G, x, w1, b1r, w2, b2r)

    out = pl.pallas_call(
        _stage2_body,
        out_shape=jax.ShapeDtypeStruct((N, K), jnp.float32),
        grid=(N // tm,),
        in_specs=[
            pl.BlockSpec((tm, N), lambda i: (i, 0)),
            pl.BlockSpec((N, K), lambda i: (0, 0)),
        ],
        out_specs=pl.BlockSpec((tm, K), lambda i: (i, 0)),
        compiler_params=pltpu.CompilerParams(
            dimension_semantics=("parallel",),
            vmem_limit_bytes=48 * 1024 * 1024,
        ),
    )(g16, p2)

    return out


# speedup vs baseline: 1.0532x; 1.0532x over previous
import jax
import jax.numpy as jnp
from jax.experimental import pallas as pl
from jax.experimental.pallas import tpu as pltpu

_TM = 512


def _stage1_body(g_ref, x_ref, w1_ref, b1_ref, w2_ref, b2_ref, p2_ref):
    g = g_ref[...]
    m1 = jnp.dot(g, x_ref[...], preferred_element_type=jnp.float32)
    rs = jnp.sum(g, axis=1, keepdims=True)
    h = jnp.dot(m1, w1_ref[...], preferred_element_type=jnp.float32)
    h = jnp.maximum(h + rs * b1_ref[...], 0.0)
    p2 = jnp.dot(h, w2_ref[...], preferred_element_type=jnp.float32)
    p2_ref[...] = p2 + b2_ref[...]


def _stage2_body(g_ref, p2_ref, o_ref):
    o_ref[...] = jnp.dot(g_ref[...], p2_ref[...],
                         preferred_element_type=jnp.float32)


def kernel(x, G, w1, b1, w2, b2):
    N, C = x.shape
    H = w1.shape[1]
    K = w2.shape[1]
    tm = _TM
    b1r = b1.reshape(1, H)
    b2r = b2.reshape(1, K)

    p2 = pl.pallas_call(
        _stage1_body,
        out_shape=jax.ShapeDtypeStruct((N, K), jnp.float32),
        grid=(N // tm,),
        in_specs=[
            pl.BlockSpec((tm, N), lambda i: (i, 0)),
            pl.BlockSpec((N, C), lambda i: (0, 0)),
            pl.BlockSpec((C, H), lambda i: (0, 0)),
            pl.BlockSpec((1, H), lambda i: (0, 0)),
            pl.BlockSpec((H, K), lambda i: (0, 0)),
            pl.BlockSpec((1, K), lambda i: (0, 0)),
        ],
        out_specs=pl.BlockSpec((tm, K), lambda i: (i, 0)),
        compiler_params=pltpu.CompilerParams(
            dimension_semantics=("parallel",),
            vmem_limit_bytes=48 * 1024 * 1024,
        ),
    )(G, x, w1, b1r, w2, b2r)

    out = pl.pallas_call(
        _stage2_body,
        out_shape=jax.ShapeDtypeStruct((N, K), jnp.float32),
        grid=(N // tm,),
        in_specs=[
            pl.BlockSpec((tm, N), lambda i: (i, 0)),
            pl.BlockSpec((N, K), lambda i: (0, 0)),
        ],
        out_specs=pl.BlockSpec((tm, K), lambda i: (i, 0)),
        compiler_params=pltpu.CompilerParams(
            dimension_semantics=("parallel",),
            vmem_limit_bytes=48 * 1024 * 1024,
        ),
    )(G, p2)

    return out
